# baseline (speedup 1.0000x reference)
import functools

import jax
import jax.numpy as jnp
from jax import lax
from jax.experimental import pallas as pl
from jax.experimental.pallas import tpu as pltpu
from jax.experimental.pallas import tpu_sc as plsc

N_R = 1_000_000
N_CL = 3
NNZ = 2_000_000

HALF = 500_000
DUMP = 3 * HALF
DUMP_MASK = 8191
ACC_WORDS = 2048 * 737

C1 = 2048
N_FULL1 = NNZ // C1
TAIL1 = NNZ - N_FULL1 * C1

C2 = 1024
N_FULL2 = HALF // C2
TAIL2 = HALF - N_FULL2 * C2

_mesh = plsc.VectorSubcoreMesh(core_axis_name="c", subcore_axis_name="s")


@functools.partial(
    pl.kernel,
    out_type=[jax.ShapeDtypeStruct((N_R,), jnp.float32) for _ in range(3)],
    mesh=_mesh,
    scratch_types=[
        pltpu.VMEM_SHARED((ACC_WORDS,), jnp.float32),
        pltpu.VMEM((2048,), jnp.float32),
        pltpu.VMEM((2 * C1,), jnp.int32),
        pltpu.VMEM((2 * C1,), jnp.int32),
        pltpu.VMEM((2 * C1,), jnp.float32),
        pltpu.VMEM((C1,), jnp.int32),
        pltpu.VMEM((C1,), jnp.int32),
        pltpu.VMEM((TAIL1,), jnp.int32),
        pltpu.VMEM((144,), jnp.float32),
        pltpu.VMEM((2 * 3 * C2,), jnp.float32),
        pltpu.VMEM((2 * 3 * C2,), jnp.float32),
        pltpu.SemaphoreType.DMA,
        pltpu.SemaphoreType.DMA,
        pltpu.SemaphoreType.DMA,
        pltpu.SemaphoreType.DMA,
        pltpu.SemaphoreType.DMA,
    ],
)
def _sc_spmm(rows_hbm, cols_hbm, vals_hbm, w_hbm, o0_hbm, o1_hbm, o2_hbm,
             acc, zeros_v, rows_b, cols_b, vals_b, idx_b0, idx_b1,
             idx_t, w_v, p_b, o_b, sem_i0, sem_i1, sem_p0, sem_p1, sem_o):
    outs = (o0_hbm, o1_hbm, o2_hbm)
    idx_sets = (idx_b0, idx_b1)
    sem_is = (sem_i0, sem_i1)
    sem_ps = (sem_p0, sem_p1)
    cid = lax.axis_index("c")
    sid = lax.axis_index("s")
    row_lo = cid * HALF
    iota = lax.iota(jnp.int32, 16)

    def _start_in(k, s):
        base = k * C1
        off = s * C1
        pltpu.async_copy(rows_hbm.at[pl.ds(base, C1)],
                         rows_b.at[pl.ds(off, C1)], sem_is[s])
        pltpu.async_copy(cols_hbm.at[pl.ds(base, C1)],
                         cols_b.at[pl.ds(off, C1)], sem_is[s])
        pltpu.async_copy(vals_hbm.at[pl.ds(base, C1)],
                         vals_b.at[pl.ds(off, C1)], sem_is[s])

    def _drain_in(s):
        off = s * C1
        for ref in (rows_b, cols_b, vals_b):
            pltpu.make_async_copy(rows_hbm.at[pl.ds(0, C1)],
                                  ref.at[pl.ds(off, C1)], sem_is[s]).wait()

    _start_in(sid, 0)

    scope0 = jax.named_scope("p0_zero")
    scope0.__enter__()

    def _zfill(i, carry):
        zeros_v[pl.ds(i * 16, 16)] = jnp.zeros((16,), jnp.float32)
        return carry
    lax.fori_loop(0, 128, _zfill, 0)

    def _zchunk(m, carry):
        k = sid + m * 16

        @pl.when(k < ACC_WORDS // 2048)
        def _():
            pltpu.sync_copy(zeros_v, acc.at[pl.ds(k * 2048, 2048)])
        return carry
    lax.fori_loop(0, (ACC_WORDS // 2048 + 15) // 16, _zchunk, 0)
    plsc.subcore_barrier()
    scope0.__exit__(None, None, None)
    scope1 = jax.named_scope("p1_scatter")
    scope1.__enter__()

    def _compute_idx(s, n, iref):
        off = s * C1

        def _grp2(i, carry):
            for h in range(2):
                i16 = i * 32 + h * 16
                r = rows_b[pl.ds(off + i16, 16)]
                c = cols_b[pl.ds(off + i16, 16)]
                t = r - row_lo
                mask = (t >= 0) & (t < HALF)
                idx = c * HALF + t
                dmp = DUMP + ((i16 & DUMP_MASK) + iota)
                iref[pl.ds(i16, 16)] = jnp.where(mask, idx, dmp)
            return carry
        lax.fori_loop(0, n // 32, _grp2, 0)

    def _scatter(s, n, iref):
        pltpu.sync_copy(vals_b.at[pl.ds(s * C1, n)], acc.at[iref], add=True)

    def _p1(m, carry):
        kA = sid + m * 32
        kB = kA + 16
        kA2 = kA + 32

        @pl.when(kB < N_FULL1)
        def _():
            _start_in(kB, 1)

        @pl.when(kA < N_FULL1)
        def _():
            _drain_in(0)
            _compute_idx(0, C1, idx_b0)
            _scatter(0, C1, idx_b0)

        @pl.when(kA2 < N_FULL1)
        def _():
            _start_in(kA2, 0)

        @pl.when(kB < N_FULL1)
        def _():
            _drain_in(1)
            _compute_idx(1, C1, idx_b1)
            _scatter(1, C1, idx_b1)
        return carry
    lax.fori_loop(0, (N_FULL1 + 31) // 32, _p1, 0)

    @pl.when(sid == 15)
    def _():
        base = N_FULL1 * C1
        pltpu.sync_copy(rows_hbm.at[pl.ds(base, TAIL1)],
                        rows_b.at[pl.ds(0, TAIL1)])
        pltpu.sync_copy(cols_hbm.at[pl.ds(base, TAIL1)],
                        cols_b.at[pl.ds(0, TAIL1)])
        pltpu.sync_copy(vals_hbm.at[pl.ds(base, TAIL1)],
                        vals_b.at[pl.ds(0, TAIL1)])
        _compute_idx(0, TAIL1, idx_t)
        pltpu.sync_copy(vals_b.at[pl.ds(0, TAIL1)], acc.at[idx_t], add=True)

    plsc.subcore_barrier()
    scope1.__exit__(None, None, None)
    scope2 = jax.named_scope("p2_combine")
    scope2.__enter__()

    pltpu.sync_copy(w_hbm, w_v)
    w = [[w_v[pl.ds((3 * i + j) * 16, 16)] for j in range(3)] for i in range(3)]

    def _start_p2(k, s):
        row0 = k * C2
        for c in range(3):
            pltpu.async_copy(acc.at[pl.ds(c * HALF + row0, C2)],
                             p_b.at[pl.ds((s * 3 + c) * C2, C2)], sem_ps[s])

    def _drain_p2(s):
        for c in range(3):
            pltpu.make_async_copy(acc.at[pl.ds(0, C2)],
                                  p_b.at[pl.ds((s * 3 + c) * C2, C2)],
                                  sem_ps[s]).wait()

    def _fma(s, n):
        poff = s * 3 * C2

        def _grp(l, carry):
            l16 = l * 16
            a = [p_b[pl.ds(poff + c * C2 + l16, 16)] for c in range(3)]
            for j in range(3):
                o_b[pl.ds(poff + j * C2 + l16, 16)] = (
                    a[0] * w[0][j] + a[1] * w[1][j] + a[2] * w[2][j])
            return carry
        lax.fori_loop(0, n // 16, _grp, 0)

    def _out(k, s, n):
        row0 = row_lo + k * C2
        hs = [pltpu.async_copy(o_b.at[pl.ds((s * 3 + j) * C2, n)],
                               outs[j].at[pl.ds(row0, n)], sem_o)
              for j in range(3)]
        for h in hs:
            h.wait()

    @pl.when(sid < N_FULL2)
    def _():
        _start_p2(sid, 0)

    def _p2(m, carry):
        kA = sid + m * 32
        kB = kA + 16
        kA2 = kA + 32

        @pl.when(kB < N_FULL2)
        def _():
            _start_p2(kB, 1)

        @pl.when(kA < N_FULL2)
        def _():
            _drain_p2(0)
            _fma(0, C2)
            _out(kA, 0, C2)

        @pl.when(kA2 < N_FULL2)
        def _():
            _start_p2(kA2, 0)

        @pl.when(kB < N_FULL2)
        def _():
            _drain_p2(1)
            _fma(1, C2)
            _out(kB, 1, C2)
        return carry
    lax.fori_loop(0, (N_FULL2 + 31) // 32, _p2, 0)

    @pl.when(sid == 15)
    def _():
        row0 = N_FULL2 * C2
        for c in range(3):
            pltpu.sync_copy(acc.at[pl.ds(c * HALF + row0, TAIL2)],
                            p_b.at[pl.ds(c * C2, TAIL2)])
        _fma(0, TAIL2)
        for j in range(3):
            pltpu.sync_copy(o_b.at[pl.ds(j * C2, TAIL2)],
                            outs[j].at[pl.ds(row_lo + row0, TAIL2)])
    scope2.__exit__(None, None, None)


def kernel(x_rows, x_cols, x_values, dense_weight):
    wb = jnp.broadcast_to(dense_weight.reshape(9, 1), (9, 16)).reshape(144)
    o0, o1, o2 = _sc_spmm(x_rows.astype(jnp.int32), x_cols.astype(jnp.int32),
                          x_values, wb)
    return jnp.stack([o0, o1, o2], axis=1)


# speedup vs baseline: 1.4151x; 1.4151x over previous
import functools

import jax
import jax.numpy as jnp
from jax import lax
from jax.experimental import pallas as pl
from jax.experimental.pallas import tpu as pltpu
from jax.experimental.pallas import tpu_sc as plsc

N_R = 1_000_000
N_CL = 3
NNZ = 2_000_000

HALF = 500_000
DUMP = 3 * HALF
DUMP_MASK = 8191
ACC_WORDS = 2048 * 737

C1 = 2048
N_FULL1 = NNZ // C1
TAIL1 = NNZ - N_FULL1 * C1

C2 = 1024
N_FULL2 = HALF // C2
TAIL2 = HALF - N_FULL2 * C2

_mesh = plsc.VectorSubcoreMesh(core_axis_name="c", subcore_axis_name="s")


@functools.partial(
    pl.kernel,
    out_type=[jax.ShapeDtypeStruct((N_R,), jnp.float32) for _ in range(3)],
    mesh=_mesh,
    scratch_types=[
        pltpu.VMEM_SHARED((ACC_WORDS,), jnp.float32),
        pltpu.VMEM((2048,), jnp.float32),
        pltpu.VMEM((2 * C1,), jnp.int32),
        pltpu.VMEM((2 * C1,), jnp.int32),
        pltpu.VMEM((2 * C1,), jnp.float32),
        pltpu.VMEM((C1,), jnp.int32),
        pltpu.VMEM((C1,), jnp.int32),
        pltpu.VMEM((TAIL1,), jnp.int32),
        pltpu.VMEM((144,), jnp.float32),
        pltpu.VMEM((2 * 3 * C2,), jnp.float32),
        pltpu.VMEM((2 * 3 * C2,), jnp.float32),
        pltpu.SemaphoreType.DMA,
        pltpu.SemaphoreType.DMA,
        pltpu.SemaphoreType.DMA,
        pltpu.SemaphoreType.DMA,
        pltpu.SemaphoreType.DMA,
        pltpu.SemaphoreType.DMA,
        pltpu.SemaphoreType.DMA,
    ],
)
def _sc_spmm(rows_hbm, cols_hbm, vals_hbm, w_hbm, o0_hbm, o1_hbm, o2_hbm,
             acc, zeros_v, rows_b, cols_b, vals_b, idx_b0, idx_b1,
             idx_t, w_v, p_b, o_b, sem_i0, sem_i1, sem_s0, sem_s1,
             sem_p0, sem_p1, sem_o):
    outs = (o0_hbm, o1_hbm, o2_hbm)
    idx_sets = (idx_b0, idx_b1)
    sem_is = (sem_i0, sem_i1)
    sem_ss = (sem_s0, sem_s1)
    sem_ps = (sem_p0, sem_p1)
    cid = lax.axis_index("c")
    sid = lax.axis_index("s")
    row_lo = cid * HALF
    iota = lax.iota(jnp.int32, 16)

    def _start_in(k, s):
        base = k * C1
        off = s * C1
        pltpu.async_copy(rows_hbm.at[pl.ds(base, C1)],
                         rows_b.at[pl.ds(off, C1)], sem_is[s])
        pltpu.async_copy(cols_hbm.at[pl.ds(base, C1)],
                         cols_b.at[pl.ds(off, C1)], sem_is[s])
        pltpu.async_copy(vals_hbm.at[pl.ds(base, C1)],
                         vals_b.at[pl.ds(off, C1)], sem_is[s])

    def _drain_in(s):
        off = s * C1
        for ref in (rows_b, cols_b, vals_b):
            pltpu.make_async_copy(rows_hbm.at[pl.ds(0, C1)],
                                  ref.at[pl.ds(off, C1)], sem_is[s]).wait()

    _start_in(sid, 0)

    scope0 = jax.named_scope("p0_zero")
    scope0.__enter__()

    def _zfill(i, carry):
        zeros_v[pl.ds(i * 16, 16)] = jnp.zeros((16,), jnp.float32)
        return carry
    lax.fori_loop(0, 128, _zfill, 0)

    def _zchunk(m, carry):
        k = sid + m * 16

        @pl.when(k < ACC_WORDS // 2048)
        def _():
            pltpu.sync_copy(zeros_v, acc.at[pl.ds(k * 2048, 2048)])
        return carry
    lax.fori_loop(0, (ACC_WORDS // 2048 + 15) // 16, _zchunk, 0)
    plsc.subcore_barrier()
    scope0.__exit__(None, None, None)
    scope1 = jax.named_scope("p1_scatter")
    scope1.__enter__()

    def _compute_idx(s, n, iref):
        off = s * C1

        def _grp4(i, carry):
            for h in range(4):
                i16 = i * 64 + h * 16
                r = rows_b[pl.ds(off + i16, 16)]
                c = cols_b[pl.ds(off + i16, 16)]
                t = r - row_lo
                mask = (t >= 0) & (t < HALF)
                idx = c * HALF + t
                dmp = DUMP + ((i16 & DUMP_MASK) + iota)
                iref[pl.ds(i16, 16)] = jnp.where(mask, idx, dmp)
            return carry
        lax.fori_loop(0, n // 64, _grp4, 0)

    def _scatter_start(s, iref):
        pltpu.async_copy(vals_b.at[pl.ds(s * C1, C1)], acc.at[iref],
                         sem_ss[s], add=True)

    def _wait_sc(s):
        pltpu.make_async_copy(vals_b.at[pl.ds(s * C1, C1)],
                              acc.at[idx_sets[s]], sem_ss[s]).wait()

    def _p1(m, carry):
        kA = sid + m * 32
        kB = kA + 16
        kA2 = kA + 32

        @pl.when(m > 0)
        def _():
            _wait_sc(1)

        @pl.when(kB < N_FULL1)
        def _():
            _start_in(kB, 1)

        _drain_in(0)
        _compute_idx(0, C1, idx_b0)
        _scatter_start(0, idx_b0)

        @pl.when(kB < N_FULL1)
        def _():
            _drain_in(1)
            _compute_idx(1, C1, idx_b1)
            _wait_sc(0)

            @pl.when(kA2 < N_FULL1)
            def _():
                _start_in(kA2, 0)
            _scatter_start(1, idx_b1)

        @pl.when(kB >= N_FULL1)
        def _():
            _wait_sc(0)
        return carry
    lax.fori_loop(0, (N_FULL1 + 31) // 32, _p1, 0)

    @pl.when(sid == 15)
    def _():
        base = N_FULL1 * C1
        pltpu.sync_copy(rows_hbm.at[pl.ds(base, TAIL1)],
                        rows_b.at[pl.ds(0, TAIL1)])
        pltpu.sync_copy(cols_hbm.at[pl.ds(base, TAIL1)],
                        cols_b.at[pl.ds(0, TAIL1)])
        pltpu.sync_copy(vals_hbm.at[pl.ds(base, TAIL1)],
                        vals_b.at[pl.ds(0, TAIL1)])
        _compute_idx(0, TAIL1, idx_t)
        pltpu.sync_copy(vals_b.at[pl.ds(0, TAIL1)], acc.at[idx_t], add=True)

    plsc.subcore_barrier()
    scope1.__exit__(None, None, None)
    scope2 = jax.named_scope("p2_combine")
    scope2.__enter__()

    pltpu.sync_copy(w_hbm, w_v)
    w = [[w_v[pl.ds((3 * i + j) * 16, 16)] for j in range(3)] for i in range(3)]

    def _start_p2(k, s):
        row0 = k * C2
        for c in range(3):
            pltpu.async_copy(acc.at[pl.ds(c * HALF + row0, C2)],
                             p_b.at[pl.ds((s * 3 + c) * C2, C2)], sem_ps[s])

    def _drain_p2(s):
        for c in range(3):
            pltpu.make_async_copy(acc.at[pl.ds(0, C2)],
                                  p_b.at[pl.ds((s * 3 + c) * C2, C2)],
                                  sem_ps[s]).wait()

    def _fma(s, n):
        poff = s * 3 * C2

        def _grp(l, carry):
            l16 = l * 16
            a = [p_b[pl.ds(poff + c * C2 + l16, 16)] for c in range(3)]
            for j in range(3):
                o_b[pl.ds(poff + j * C2 + l16, 16)] = (
                    a[0] * w[0][j] + a[1] * w[1][j] + a[2] * w[2][j])
            return carry
        lax.fori_loop(0, n // 16, _grp, 0)

    def _out(k, s, n):
        row0 = row_lo + k * C2
        hs = [pltpu.async_copy(o_b.at[pl.ds((s * 3 + j) * C2, n)],
                               outs[j].at[pl.ds(row0, n)], sem_o)
              for j in range(3)]
        for h in hs:
            h.wait()

    @pl.when(sid < N_FULL2)
    def _():
        _start_p2(sid, 0)

    def _p2(m, carry):
        kA = sid + m * 32
        kB = kA + 16
        kA2 = kA + 32

        @pl.when(kB < N_FULL2)
        def _():
            _start_p2(kB, 1)

        @pl.when(kA < N_FULL2)
        def _():
            _drain_p2(0)
            _fma(0, C2)
            _out(kA, 0, C2)

        @pl.when(kA2 < N_FULL2)
        def _():
            _start_p2(kA2, 0)

        @pl.when(kB < N_FULL2)
        def _():
            _drain_p2(1)
            _fma(1, C2)
            _out(kB, 1, C2)
        return carry
    lax.fori_loop(0, (N_FULL2 + 31) // 32, _p2, 0)

    @pl.when(sid == 15)
    def _():
        row0 = N_FULL2 * C2
        for c in range(3):
            pltpu.sync_copy(acc.at[pl.ds(c * HALF + row0, TAIL2)],
                            p_b.at[pl.ds(c * C2, TAIL2)])
        _fma(0, TAIL2)
        for j in range(3):
            pltpu.sync_copy(o_b.at[pl.ds(j * C2, TAIL2)],
                            outs[j].at[pl.ds(row_lo + row0, TAIL2)])
    scope2.__exit__(None, None, None)


def kernel(x_rows, x_cols, x_values, dense_weight):
    wb = jnp.broadcast_to(dense_weight.reshape(9, 1), (9, 16)).reshape(144)
    o0, o1, o2 = _sc_spmm(x_rows.astype(jnp.int32), x_cols.astype(jnp.int32),
                          x_values, wb)
    e = jnp.eye(3, dtype=jnp.float32)
    return o0[:, None] * e[0] + o1[:, None] * e[1] + o2[:, None] * e[2]


# speedup vs baseline: 1.4395x; 1.0173x over previous
import functools

import jax
import jax.numpy as jnp
from jax import lax
from jax.experimental import pallas as pl
from jax.experimental.pallas import tpu as pltpu
from jax.experimental.pallas import tpu_sc as plsc

N_R = 1_000_000
N_CL = 3
NNZ = 2_000_000

HALF = 500_000
DUMP = 3 * HALF
DUMP_MASK = 8191
ACC_WORDS = 2048 * 737

C1 = 2048
N_FULL1 = NNZ // C1
TAIL1 = NNZ - N_FULL1 * C1

C2 = 1024
N_FULL2 = HALF // C2
TAIL2 = HALF - N_FULL2 * C2

_mesh = plsc.VectorSubcoreMesh(core_axis_name="c", subcore_axis_name="s")


@functools.partial(
    pl.kernel,
    out_type=[jax.ShapeDtypeStruct((N_R,), jnp.float32) for _ in range(3)],
    mesh=_mesh,
    scratch_types=[
        pltpu.VMEM_SHARED((ACC_WORDS,), jnp.float32),
        pltpu.VMEM((2048,), jnp.float32),
        pltpu.VMEM((2 * C1,), jnp.int32),
        pltpu.VMEM((2 * C1,), jnp.int32),
        pltpu.VMEM((2 * C1,), jnp.float32),
        pltpu.VMEM((C1,), jnp.int32),
        pltpu.VMEM((C1,), jnp.int32),
        pltpu.VMEM((TAIL1,), jnp.int32),
        pltpu.VMEM((144,), jnp.float32),
        pltpu.VMEM((2 * 3 * C2,), jnp.float32),
        pltpu.VMEM((2 * 3 * C2,), jnp.float32),
        pltpu.SemaphoreType.DMA,
        pltpu.SemaphoreType.DMA,
        pltpu.SemaphoreType.DMA,
        pltpu.SemaphoreType.DMA,
        pltpu.SemaphoreType.DMA,
        pltpu.SemaphoreType.DMA,
        pltpu.SemaphoreType.DMA,
    ],
)
def _sc_spmm(rows_hbm, cols_hbm, vals_hbm, w_hbm, o0_hbm, o1_hbm, o2_hbm,
             acc, zeros_v, rows_b, cols_b, vals_b, idx_b0, idx_b1,
             idx_t, w_v, p_b, o_b, sem_i0, sem_i1, sem_s0, sem_s1,
             sem_p0, sem_p1, sem_o):
    outs = (o0_hbm, o1_hbm, o2_hbm)
    idx_sets = (idx_b0, idx_b1)
    sem_is = (sem_i0, sem_i1)
    sem_ss = (sem_s0, sem_s1)
    sem_ps = (sem_p0, sem_p1)
    cid = lax.axis_index("c")
    sid = lax.axis_index("s")
    row_lo = cid * HALF
    iota = lax.iota(jnp.int32, 16)

    def _start_in(k, s):
        base = k * C1
        off = s * C1
        pltpu.async_copy(rows_hbm.at[pl.ds(base, C1)],
                         rows_b.at[pl.ds(off, C1)], sem_is[s])
        pltpu.async_copy(cols_hbm.at[pl.ds(base, C1)],
                         cols_b.at[pl.ds(off, C1)], sem_is[s])
        pltpu.async_copy(vals_hbm.at[pl.ds(base, C1)],
                         vals_b.at[pl.ds(off, C1)], sem_is[s])

    def _drain_in(s):
        off = s * C1
        for ref in (rows_b, cols_b, vals_b):
            pltpu.make_async_copy(rows_hbm.at[pl.ds(0, C1)],
                                  ref.at[pl.ds(off, C1)], sem_is[s]).wait()

    _start_in(sid, 0)

    def _zfill(i, carry):
        zeros_v[pl.ds(i * 16, 16)] = jnp.zeros((16,), jnp.float32)
        return carry
    lax.fori_loop(0, 128, _zfill, 0)
    n_zc = ACC_WORDS // 2048

    def _zchunk(m, carry):
        k = sid + m * 16

        @pl.when(k < n_zc)
        def _():
            pltpu.async_copy(zeros_v, acc.at[pl.ds(k * 2048, 2048)], sem_o)
        return carry
    lax.fori_loop(0, (n_zc + 15) // 16, _zchunk, 0)

    def _zdrain(m, carry):
        k = sid + m * 16

        @pl.when(k < n_zc)
        def _():
            pltpu.make_async_copy(zeros_v, acc.at[pl.ds(0, 2048)],
                                  sem_o).wait()
        return carry
    lax.fori_loop(0, (n_zc + 15) // 16, _zdrain, 0)
    plsc.subcore_barrier()

    def _compute_idx(s, n, iref):
        off = s * C1

        def _grp4(i, carry):
            for h in range(4):
                i16 = i * 64 + h * 16
                r = rows_b[pl.ds(off + i16, 16)]
                c = cols_b[pl.ds(off + i16, 16)]
                t = r - row_lo
                mask = (t >= 0) & (t < HALF)
                idx = c * HALF + t
                dmp = DUMP + ((i16 & DUMP_MASK) + iota)
                iref[pl.ds(i16, 16)] = jnp.where(mask, idx, dmp)
            return carry
        lax.fori_loop(0, n // 64, _grp4, 0)

    def _scatter_start(s, iref):
        pltpu.async_copy(vals_b.at[pl.ds(s * C1, C1)], acc.at[iref],
                         sem_ss[s], add=True)

    def _wait_sc(s):
        pltpu.make_async_copy(vals_b.at[pl.ds(s * C1, C1)],
                              acc.at[idx_sets[s]], sem_ss[s]).wait()

    def _p1(m, carry):
        kA = sid + m * 32
        kB = kA + 16
        kA2 = kA + 32

        @pl.when(m > 0)
        def _():
            _wait_sc(1)

        @pl.when(kB < N_FULL1)
        def _():
            _start_in(kB, 1)

        _drain_in(0)
        _compute_idx(0, C1, idx_b0)
        _scatter_start(0, idx_b0)

        @pl.when(kB < N_FULL1)
        def _():
            _drain_in(1)
            _compute_idx(1, C1, idx_b1)
            _wait_sc(0)

            @pl.when(kA2 < N_FULL1)
            def _():
                _start_in(kA2, 0)
            _scatter_start(1, idx_b1)

        @pl.when(kB >= N_FULL1)
        def _():
            _wait_sc(0)
        return carry
    lax.fori_loop(0, (N_FULL1 + 31) // 32, _p1, 0)

    @pl.when(sid == 15)
    def _():
        base = N_FULL1 * C1
        pltpu.sync_copy(rows_hbm.at[pl.ds(base, TAIL1)],
                        rows_b.at[pl.ds(0, TAIL1)])
        pltpu.sync_copy(cols_hbm.at[pl.ds(base, TAIL1)],
                        cols_b.at[pl.ds(0, TAIL1)])
        pltpu.sync_copy(vals_hbm.at[pl.ds(base, TAIL1)],
                        vals_b.at[pl.ds(0, TAIL1)])
        _compute_idx(0, TAIL1, idx_t)
        pltpu.sync_copy(vals_b.at[pl.ds(0, TAIL1)], acc.at[idx_t], add=True)

    plsc.subcore_barrier()

    pltpu.sync_copy(w_hbm, w_v)
    w = [[w_v[pl.ds((3 * i + j) * 16, 16)] for j in range(3)] for i in range(3)]

    def _start_p2(k, s):
        row0 = k * C2
        for c in range(3):
            pltpu.async_copy(acc.at[pl.ds(c * HALF + row0, C2)],
                             p_b.at[pl.ds((s * 3 + c) * C2, C2)], sem_ps[s])

    def _drain_p2(s):
        for c in range(3):
            pltpu.make_async_copy(acc.at[pl.ds(0, C2)],
                                  p_b.at[pl.ds((s * 3 + c) * C2, C2)],
                                  sem_ps[s]).wait()

    def _fma(s, n):
        poff = s * 3 * C2

        def _grp(l, carry):
            l16 = l * 16
            a = [p_b[pl.ds(poff + c * C2 + l16, 16)] for c in range(3)]
            for j in range(3):
                o_b[pl.ds(poff + j * C2 + l16, 16)] = (
                    a[0] * w[0][j] + a[1] * w[1][j] + a[2] * w[2][j])
            return carry
        lax.fori_loop(0, n // 16, _grp, 0)

    def _out(k, s, n):
        row0 = row_lo + k * C2
        hs = [pltpu.async_copy(o_b.at[pl.ds((s * 3 + j) * C2, n)],
                               outs[j].at[pl.ds(row0, n)], sem_o)
              for j in range(3)]
        for h in hs:
            h.wait()

    @pl.when(sid < N_FULL2)
    def _():
        _start_p2(sid, 0)

    def _p2(m, carry):
        kA = sid + m * 32
        kB = kA + 16
        kA2 = kA + 32

        @pl.when(kB < N_FULL2)
        def _():
            _start_p2(kB, 1)

        @pl.when(kA < N_FULL2)
        def _():
            _drain_p2(0)
            _fma(0, C2)
            _out(kA, 0, C2)

        @pl.when(kA2 < N_FULL2)
        def _():
            _start_p2(kA2, 0)

        @pl.when(kB < N_FULL2)
        def _():
            _drain_p2(1)
            _fma(1, C2)
            _out(kB, 1, C2)
        return carry
    lax.fori_loop(0, (N_FULL2 + 31) // 32, _p2, 0)

    @pl.when(sid == 15)
    def _():
        row0 = N_FULL2 * C2
        for c in range(3):
            pltpu.sync_copy(acc.at[pl.ds(c * HALF + row0, TAIL2)],
                            p_b.at[pl.ds(c * C2, TAIL2)])
        _fma(0, TAIL2)
        for j in range(3):
            pltpu.sync_copy(o_b.at[pl.ds(j * C2, TAIL2)],
                            outs[j].at[pl.ds(row_lo + row0, TAIL2)])


def kernel(x_rows, x_cols, x_values, dense_weight):
    wb = jnp.broadcast_to(dense_weight.reshape(9, 1), (9, 16)).reshape(144)
    o0, o1, o2 = _sc_spmm(x_rows.astype(jnp.int32), x_cols.astype(jnp.int32),
                          x_values, wb)
    e = jnp.eye(3, dtype=jnp.float32)
    return o0[:, None] * e[0] + o1[:, None] * e[1] + o2[:, None] * e[2]
